# baseline (speedup 1.0000x reference)
import functools

import jax
import jax.numpy as jnp
from jax import lax
from jax.experimental import pallas as pl
from jax.experimental.pallas import tpu as pltpu
from jax.experimental.pallas import tpu_sc as plsc

NC, NS = 2, 16
NW = NC * NS
C = 128
NJ = 200
NBUF = 4
ROWS_W = NJ * C
BATCH = 4096
HIST = 200
DIM = 64


def _gather_body(table_hbm, idx_hbm, out_hbm, idx_v, rows_v, gsems, wsems):
    wid = lax.axis_index("s") * NC + lax.axis_index("c")
    base = wid * ROWS_W

    pltpu.sync_copy(idx_hbm.at[wid], idx_v)

    def start_gather(j, b):
        pltpu.async_copy(table_hbm.at[idx_v.at[j]], rows_v.at[b], gsems.at[b])

    def wait_gather(b):
        pltpu.make_async_copy(
            table_hbm.at[idx_v.at[0]], rows_v.at[b], gsems.at[b]
        ).wait()

    def start_write(j, b):
        pltpu.async_copy(
            rows_v.at[b], out_hbm.at[pl.ds(base + j * C, C)], wsems.at[b]
        )

    def wait_write(b):
        pltpu.make_async_copy(
            rows_v.at[b], out_hbm.at[pl.ds(base, C)], wsems.at[b]
        ).wait()

    def step(j, b, first, last):
        bg = (b - 1) % NBUF
        if not last:
            if not first:
                wait_write(bg)
            start_gather(j + NBUF - 1, bg)
        wait_gather(b)
        start_write(j, b)

    for b in range(NBUF - 1):
        start_gather(b, b)

    for b in range(NBUF):
        step(b, b, first=(b == 0), last=False)

    @pl.loop(1, NJ // NBUF - 1)
    def _(jj):
        j0 = jj * NBUF
        for b in range(NBUF):
            step(j0 + b, b, first=False, last=False)

    j0 = NJ - NBUF
    for b in range(NBUF):
        step(j0 + b, b, first=False, last=(b != 0))

    for b in range(NBUF):
        wait_write(b)


_gather = pl.kernel(
    _gather_body,
    out_type=jax.ShapeDtypeStruct((NW * ROWS_W, DIM), jnp.float32),
    mesh=plsc.VectorSubcoreMesh(
        core_axis_name="c", subcore_axis_name="s", num_cores=NC, num_subcores=NS
    ),
    scratch_types=[
        pltpu.VMEM((NJ, C), jnp.int32),
        pltpu.VMEM((NBUF, C, DIM), jnp.float32),
        pltpu.SemaphoreType.DMA((NBUF,)),
        pltpu.SemaphoreType.DMA((NBUF,)),
    ],
    compiler_params=pltpu.CompilerParams(use_tc_tiling_on_sc=False),
)


def kernel(x, table):
    idx = x.reshape(NW, NJ, C)
    out = _gather(table, idx)
    return out.reshape(BATCH, HIST, DIM)
